# baseline (speedup 1.0000x reference)
import functools

import jax
import jax.numpy as jnp
from jax.experimental import pallas as pl
from jax.experimental.pallas import tpu as pltpu

HIDDEN = 2048
NUM_EXPERTS = 16
TOP_K = 2
BETA = 0.1
GAMMA = 0.1
EXPLORATION_C = 0.1

BLK = 2048


def _router_body(x_ref, wt_ref, rep_ref, loads_ref, cnts_ref, tot_ref,
                 out_ref, aux_ref, cnt_acc, psum_acc):
    i = pl.program_id(0)
    nsteps = pl.num_programs(0)
    logits = jnp.dot(x_ref[...], wt_ref[...],
                     preferred_element_type=jnp.float32)
    tot = tot_ref[0, 0]
    bias = (BETA * rep_ref[...] - GAMMA * loads_ref[...]
            + EXPLORATION_C * jnp.sqrt(
                jnp.log(tot + 1.0) / (cnts_ref[...] + 1e-10)))
    s = logits + bias

    u = jax.lax.bitcast_convert_type(s, jnp.int32)
    key = jnp.where(u < 0, u ^ jnp.int32(0x7FFFFFFF), u)
    eids = jax.lax.broadcasted_iota(jnp.int32, (BLK, NUM_EXPERTS), 1)
    key = (key & jnp.int32(~0xF)) | (jnp.int32(NUM_EXPERTS - 1) - eids)

    m1k = jnp.max(key, axis=1, keepdims=True)
    key2 = jnp.where(key == m1k, jnp.int32(-2147483648), key)
    m2k = jnp.max(key2, axis=1, keepdims=True)

    i1 = jnp.int32(NUM_EXPERTS - 1) - (m1k & jnp.int32(0xF))
    i2 = jnp.int32(NUM_EXPERTS - 1) - (m2k & jnp.int32(0xF))
    u1 = jnp.where(m1k < 0, m1k ^ jnp.int32(0x7FFFFFFF), m1k)
    u2 = jnp.where(m2k < 0, m2k ^ jnp.int32(0x7FFFFFFF), m2k)
    s1 = jax.lax.bitcast_convert_type(u1, jnp.float32)
    s2 = jax.lax.bitcast_convert_type(u2, jnp.float32)

    e2 = jnp.exp(s2 - s1)
    w1 = 1.0 / (1.0 + e2)
    i1f = jax.lax.bitcast_convert_type(i1, jnp.float32)
    i2f = jax.lax.bitcast_convert_type(i2, jnp.float32)
    pad = jnp.zeros((BLK, NUM_EXPERTS - 4), jnp.float32)
    out_ref[...] = jnp.concatenate([w1, 1.0 - w1, i1f, i2f, pad], axis=1)

    z = jnp.exp(s - s1)
    probs = z * (1.0 / jnp.sum(z, axis=1, keepdims=True))
    oh = ((key == m1k).astype(jnp.float32)
          + (key == m2k).astype(jnp.float32))

    @pl.when(i == 0)
    def _init():
        cnt_acc[...] = jnp.zeros_like(cnt_acc)
        psum_acc[...] = jnp.zeros_like(psum_acc)

    cnt_acc[...] += jnp.sum(oh, axis=0, keepdims=True)
    psum_acc[...] += jnp.sum(probs, axis=0, keepdims=True)

    @pl.when(i == nsteps - 1)
    def _fin():
        t_total = jnp.float32(BLK * nsteps)
        aux_ref[...] = (jnp.sum(cnt_acc[...] * psum_acc[...], keepdims=True)
                        * (NUM_EXPERTS / (t_total * t_total)))


@functools.partial(jax.jit, static_argnames=("interpret",))
def _run(x, W, reputation_scores, expert_loads, expert_counts,
         total_routing_decisions, interpret=False):
    B, S, H = x.shape
    T = B * S
    nsteps = T // BLK
    x2 = x.reshape(T, H)
    wt = W.T
    rep = reputation_scores.reshape(1, NUM_EXPERTS)
    loads = expert_loads.reshape(1, NUM_EXPERTS)
    cnts = expert_counts.reshape(1, NUM_EXPERTS)
    tot = total_routing_decisions.reshape(1, 1)

    packed, aux = pl.pallas_call(
        _router_body,
        grid=(nsteps,),
        in_specs=[
            pl.BlockSpec((BLK, HIDDEN), lambda i: (i, 0)),
            pl.BlockSpec((HIDDEN, NUM_EXPERTS), lambda i: (0, 0)),
            pl.BlockSpec((1, NUM_EXPERTS), lambda i: (0, 0)),
            pl.BlockSpec((1, NUM_EXPERTS), lambda i: (0, 0)),
            pl.BlockSpec((1, NUM_EXPERTS), lambda i: (0, 0)),
            pl.BlockSpec((1, 1), lambda i: (0, 0)),
        ],
        out_specs=[
            pl.BlockSpec((BLK, NUM_EXPERTS), lambda i: (i, 0)),
            pl.BlockSpec((1, 1), lambda i: (0, 0)),
        ],
        out_shape=[
            jax.ShapeDtypeStruct((T, NUM_EXPERTS), jnp.float32),
            jax.ShapeDtypeStruct((1, 1), jnp.float32),
        ],
        scratch_shapes=[
            pltpu.VMEM((1, NUM_EXPERTS), jnp.float32),
            pltpu.VMEM((1, NUM_EXPERTS), jnp.float32),
        ],
        interpret=interpret,
    )(x2, wt, rep, loads, cnts, tot)
    w_flat = packed[:, :TOP_K]
    idx_flat = jax.lax.bitcast_convert_type(
        packed[:, TOP_K:2 * TOP_K], jnp.int32)
    return (w_flat.reshape(B, S, TOP_K),
            idx_flat.reshape(B, S, TOP_K),
            aux[0, 0])


def kernel(x, W, reputation_scores, expert_loads, expert_counts,
           total_routing_decisions):
    return _run(x, W, reputation_scores, expert_loads, expert_counts,
                total_routing_decisions)


# speedup vs baseline: 1.0936x; 1.0936x over previous
import functools

import jax
import jax.numpy as jnp
from jax.experimental import pallas as pl
from jax.experimental.pallas import tpu as pltpu

HIDDEN = 2048
NUM_EXPERTS = 16
TOP_K = 2
BETA = 0.1
GAMMA = 0.1
EXPLORATION_C = 0.1

BLK = 2048


def _router_body(x_ref, wt_ref, rep_ref, loads_ref, cnts_ref, tot_ref,
                 w_ref, idx_ref, aux_ref, cnt_acc, psum_acc):
    i = pl.program_id(0)
    nsteps = pl.num_programs(0)
    logits = jnp.dot(x_ref[...], wt_ref[...],
                     preferred_element_type=jnp.float32)
    tot = tot_ref[0, 0]
    bias = (BETA * rep_ref[...] - GAMMA * loads_ref[...]
            + EXPLORATION_C * jnp.sqrt(
                jnp.log(tot + 1.0) / (cnts_ref[...] + 1e-10)))
    s = logits + bias
    iota = jax.lax.broadcasted_iota(jnp.int32, (BLK, NUM_EXPERTS), 1)
    m1 = jnp.max(s, axis=1, keepdims=True)
    i1 = jnp.min(jnp.where(s == m1, iota, NUM_EXPERTS), axis=1, keepdims=True)
    s2 = jnp.where(iota == i1, -1e30, s)
    m2 = jnp.max(s2, axis=1, keepdims=True)
    i2 = jnp.min(jnp.where(s2 == m2, iota, NUM_EXPERTS), axis=1, keepdims=True)
    e2 = jnp.exp(m2 - m1)
    w1 = 1.0 / (1.0 + e2)
    wpad = jnp.zeros((BLK, NUM_EXPERTS - TOP_K), jnp.float32)
    w_ref[...] = jnp.concatenate([w1, 1.0 - w1, wpad], axis=1)
    ipad = jnp.zeros((BLK, NUM_EXPERTS - TOP_K), jnp.int32)
    idx_ref[...] = jnp.concatenate([i1, i2, ipad], axis=1)
    z = jnp.exp(s - m1)
    probs = z / jnp.sum(z, axis=1, keepdims=True)
    oh = ((iota == i1).astype(jnp.float32)
          + (iota == i2).astype(jnp.float32))

    @pl.when(i == 0)
    def _init():
        cnt_acc[...] = jnp.zeros_like(cnt_acc)
        psum_acc[...] = jnp.zeros_like(psum_acc)

    cnt_acc[...] += jnp.sum(oh, axis=0, keepdims=True)
    psum_acc[...] += jnp.sum(probs, axis=0, keepdims=True)

    @pl.when(i == nsteps - 1)
    def _fin():
        t_total = jnp.float32(BLK * nsteps)
        aux_ref[...] = (jnp.sum(cnt_acc[...] * psum_acc[...], keepdims=True)
                        * (NUM_EXPERTS / (t_total * t_total)))


@functools.partial(jax.jit, static_argnames=("interpret",))
def _run(x, W, reputation_scores, expert_loads, expert_counts,
         total_routing_decisions, interpret=False):
    B, S, H = x.shape
    T = B * S
    nsteps = T // BLK
    x2 = x.reshape(T, H)
    wt = W.T
    rep = reputation_scores.reshape(1, NUM_EXPERTS)
    loads = expert_loads.reshape(1, NUM_EXPERTS)
    cnts = expert_counts.reshape(1, NUM_EXPERTS)
    tot = total_routing_decisions.reshape(1, 1)

    w_wide, idx_wide, aux = pl.pallas_call(
        _router_body,
        grid=(nsteps,),
        in_specs=[
            pl.BlockSpec((BLK, HIDDEN), lambda i: (i, 0)),
            pl.BlockSpec((HIDDEN, NUM_EXPERTS), lambda i: (0, 0)),
            pl.BlockSpec((1, NUM_EXPERTS), lambda i: (0, 0)),
            pl.BlockSpec((1, NUM_EXPERTS), lambda i: (0, 0)),
            pl.BlockSpec((1, NUM_EXPERTS), lambda i: (0, 0)),
            pl.BlockSpec((1, 1), lambda i: (0, 0)),
        ],
        out_specs=[
            pl.BlockSpec((BLK, NUM_EXPERTS), lambda i: (i, 0)),
            pl.BlockSpec((BLK, NUM_EXPERTS), lambda i: (i, 0)),
            pl.BlockSpec((1, 1), lambda i: (0, 0)),
        ],
        out_shape=[
            jax.ShapeDtypeStruct((T, NUM_EXPERTS), jnp.float32),
            jax.ShapeDtypeStruct((T, NUM_EXPERTS), jnp.int32),
            jax.ShapeDtypeStruct((1, 1), jnp.float32),
        ],
        scratch_shapes=[
            pltpu.VMEM((1, NUM_EXPERTS), jnp.float32),
            pltpu.VMEM((1, NUM_EXPERTS), jnp.float32),
        ],
        interpret=interpret,
    )(x2, wt, rep, loads, cnts, tot)
    return (w_wide[:, :TOP_K].reshape(B, S, TOP_K),
            idx_wide[:, :TOP_K].reshape(B, S, TOP_K),
            aux[0, 0])


def kernel(x, W, reputation_scores, expert_loads, expert_counts,
           total_routing_decisions):
    return _run(x, W, reputation_scores, expert_loads, expert_counts,
                total_routing_decisions)
